# baseline (speedup 1.0000x reference)
import functools

import jax
import jax.numpy as jnp
from jax import lax
from jax.experimental import pallas as pl
from jax.experimental.pallas import tpu as pltpu
from jax.experimental.pallas import tpu_sc as plsc

_D = 768
_NC = 2
_NS = 16
_NW = _NC * _NS
_CH = 16
_NBUF = 8


def _sc_embedding_lookup(sequence, wte, batch, seq):
    bpw = (batch * seq) // _NW
    wpb = seq // bpw
    nch = bpw // _CH
    mesh = plsc.VectorSubcoreMesh(core_axis_name="c", subcore_axis_name="s")

    @functools.partial(
        pl.kernel,
        mesh=mesh,
        out_type=jax.ShapeDtypeStruct((batch, seq, _D), jnp.float32),
        scratch_types=[
            pltpu.VMEM((bpw,), jnp.int32),
        ] + [pltpu.VMEM((_CH, _D), jnp.float32) for _ in range(_NBUF)]
          + [pltpu.SemaphoreType.DMA for _ in range(2 * _NBUF)],
    )
    def body(idx_hbm, table_hbm, out_hbm, idx_v, *bufs_sems):
        rows = bufs_sems[:_NBUF]
        gsems = bufs_sems[_NBUF:2 * _NBUF]
        ssems = bufs_sems[2 * _NBUF:]

        wid = lax.axis_index("s") * _NC + lax.axis_index("c")
        b = wid // wpb
        col = (wid % wpb) * bpw
        pltpu.sync_copy(idx_hbm.at[b, pl.ds(col, bpw)], idx_v)

        for c in range(_NBUF):
            pltpu.async_copy(
                table_hbm.at[idx_v.at[pl.ds(c * _CH, _CH)]],
                rows[c], gsems[c])

        for c in range(nch):
            p = c % _NBUF
            pltpu.make_async_copy(
                table_hbm.at[idx_v.at[pl.ds(c * _CH, _CH)]],
                rows[p], gsems[p]).wait()
            pltpu.async_copy(
                rows[p], out_hbm.at[b, pl.ds(col + c * _CH, _CH)], ssems[p])
            n = c + _NBUF
            if n < nch:
                pltpu.make_async_copy(
                    rows[p], out_hbm.at[b, pl.ds(col, _CH)], ssems[p]).wait()
                pltpu.async_copy(
                    table_hbm.at[idx_v.at[pl.ds(n * _CH, _CH)]],
                    rows[p], gsems[p])

        for c in range(max(0, nch - _NBUF), nch):
            p = c % _NBUF
            pltpu.make_async_copy(
                rows[p], out_hbm.at[b, pl.ds(col, _CH)], ssems[p]).wait()

    return body(sequence, wte)


def kernel(sequence, wte):
    batch, seq = sequence.shape
    return _sc_embedding_lookup(sequence.astype(jnp.int32), wte, batch, seq)


# speedup vs baseline: 1.0256x; 1.0256x over previous
import functools

import jax
import jax.numpy as jnp
from jax import lax
from jax.experimental import pallas as pl
from jax.experimental.pallas import tpu as pltpu
from jax.experimental.pallas import tpu_sc as plsc

_D = 768
_NC = 2
_NS = 16
_NW = _NC * _NS
_CH = 32
_NBUF = 5


def _sc_embedding_lookup(sequence, wte, batch, seq):
    bpw = (batch * seq) // _NW
    wpb = seq // bpw
    nch = bpw // _CH
    mesh = plsc.VectorSubcoreMesh(core_axis_name="c", subcore_axis_name="s")

    @functools.partial(
        pl.kernel,
        mesh=mesh,
        out_type=jax.ShapeDtypeStruct((batch, seq, _D), jnp.float32),
        scratch_types=[
            pltpu.VMEM((bpw,), jnp.int32),
        ] + [pltpu.VMEM((_CH, _D), jnp.float32) for _ in range(_NBUF)]
          + [pltpu.SemaphoreType.DMA for _ in range(2 * _NBUF)],
    )
    def body(idx_hbm, table_hbm, out_hbm, idx_v, *bufs_sems):
        rows = bufs_sems[:_NBUF]
        gsems = bufs_sems[_NBUF:2 * _NBUF]
        ssems = bufs_sems[2 * _NBUF:]

        wid = lax.axis_index("s") * _NC + lax.axis_index("c")
        b = wid // wpb
        col = (wid % wpb) * bpw
        pltpu.sync_copy(idx_hbm.at[b, pl.ds(col, bpw)], idx_v)

        for c in range(_NBUF):
            pltpu.async_copy(
                table_hbm.at[idx_v.at[pl.ds(c * _CH, _CH)]],
                rows[c], gsems[c])

        for c in range(nch):
            p = c % _NBUF
            pltpu.make_async_copy(
                table_hbm.at[idx_v.at[pl.ds(c * _CH, _CH)]],
                rows[p], gsems[p]).wait()
            pltpu.async_copy(
                rows[p], out_hbm.at[b, pl.ds(col + c * _CH, _CH)], ssems[p])
            n = c + _NBUF
            if n < nch:
                pltpu.make_async_copy(
                    rows[p], out_hbm.at[b, pl.ds(col, _CH)], ssems[p]).wait()
                pltpu.async_copy(
                    table_hbm.at[idx_v.at[pl.ds(n * _CH, _CH)]],
                    rows[p], gsems[p])

        for c in range(max(0, nch - _NBUF), nch):
            p = c % _NBUF
            pltpu.make_async_copy(
                rows[p], out_hbm.at[b, pl.ds(col, _CH)], ssems[p]).wait()

    return body(sequence, wte)


def kernel(sequence, wte):
    batch, seq = sequence.shape
    return _sc_embedding_lookup(sequence.astype(jnp.int32), wte, batch, seq)
